# baseline (speedup 1.0000x reference)
import jax
import jax.numpy as jnp
from jax.experimental import pallas as pl

B = 16384
K = 16
BLK = 2048


def _body(u_ref, v_ref, dot_ref, cat_ref):
    u = u_ref[...]
    v = v_ref[...]
    dot_ref[...] = jnp.sum(u * v, axis=1)
    cat_ref[:, :K] = u
    cat_ref[:, K:] = v


def kernel(x, W, H):
    u = jnp.take(W, x[:, 0], axis=0)
    v = jnp.take(H, x[:, 1], axis=0)
    dot, cat = pl.pallas_call(
        _body,
        grid=(B // BLK,),
        in_specs=[pl.BlockSpec((BLK, K), lambda i: (i, 0)),
                  pl.BlockSpec((BLK, K), lambda i: (i, 0))],
        out_specs=[pl.BlockSpec((BLK,), lambda i: (i,)),
                   pl.BlockSpec((BLK, 2 * K), lambda i: (i, 0))],
        out_shape=[jax.ShapeDtypeStruct((B,), jnp.float32),
                   jax.ShapeDtypeStruct((B, 2 * K), jnp.float32)],
    )(u, v)
    return dot, cat
